# baseline (speedup 1.0000x reference)
import jax
import jax.numpy as jnp
from jax import lax
from jax.experimental import pallas as pl
from jax.experimental.pallas import tpu as pltpu
from jax.experimental.pallas import tpu_sc as plsc

_B = 128
_V = 100000
_SPLIT = 49920


def _sc_copy_body(src, out, sem):
    wid = lax.axis_index("s") * 2 + lax.axis_index("c")
    rb = wid % 16
    half = wid // 16

    @pl.when(half == 0)
    def _():
        pltpu.async_copy(src.at[pl.ds(rb * 8, 8), pl.ds(0, _SPLIT)],
                         out.at[pl.ds(rb * 8, 8), pl.ds(0, _SPLIT)], sem).wait()

    @pl.when(half == 1)
    def _():
        pltpu.async_copy(src.at[pl.ds(rb * 8, 8), pl.ds(_SPLIT, _V - _SPLIT)],
                         out.at[pl.ds(rb * 8, 8), pl.ds(_SPLIT, _V - _SPLIT)],
                         sem).wait()


def _sc_copy(scores):
    mesh = plsc.VectorSubcoreMesh(core_axis_name="c", subcore_axis_name="s")
    return pl.kernel(
        _sc_copy_body,
        out_type=jax.ShapeDtypeStruct((_B, _V), jnp.float32),
        mesh=mesh,
        scratch_types=[pltpu.SemaphoreType.DMA],
    )(scores)


def kernel(input_ids, scores):
    del input_ids
    return _sc_copy(scores)


# speedup vs baseline: 13.2714x; 13.2714x over previous
import jax
import jax.numpy as jnp
from jax.experimental import pallas as pl
from jax.experimental.pallas import tpu as pltpu

_EOS = 2
_B = 128
_T = 4096
_V = 100000
_ROWS = 16


def _body(ids_ref, x_ref, o_ref):
    ids = ids_ref[...]
    eos_count = jnp.sum((ids == _EOS).astype(jnp.int32), axis=1)
    eos_count_init = eos_count
    done = (eos_count - eos_count_init) >= 2
    x = x_ref[...]
    col = jax.lax.broadcasted_iota(jnp.int32, x.shape, 1)
    masked = jnp.where(col == _EOS, 0.0, float("-inf"))
    o_ref[...] = jnp.where(done[:, None], masked, x)


def kernel(input_ids, scores):
    grid = (_B // _ROWS,)
    return pl.pallas_call(
        _body,
        grid=grid,
        in_specs=[
            pl.BlockSpec((_ROWS, _T), lambda i: (i, 0)),
            pl.BlockSpec((_ROWS, _V), lambda i: (i, 0)),
        ],
        out_specs=pl.BlockSpec((_ROWS, _V), lambda i: (i, 0)),
        out_shape=jax.ShapeDtypeStruct((_B, _V), jnp.float32),
        compiler_params=pltpu.CompilerParams(
            dimension_semantics=("parallel",),
        ),
    )(input_ids.astype(jnp.int32), scores)
